# baseline (speedup 1.0000x reference)
import functools

import jax
import jax.numpy as jnp
import numpy as np
from jax.experimental import pallas as pl
from jax.experimental.pallas import tpu as pltpu

_N_BUILDING = 120
_N_STREET = 50
_N = _N_BUILDING + _N_STREET
_B = 32
_NT = _B * _N
_E = _NT * 16
_HEADS = 4
_FD = 256
_LAT = 512
_BOT = 128
_INNER = 80
_IMG = 64


def _mm_kernel(x_ref, w_ref, o_ref, *, precision):
    o_ref[...] = jnp.dot(x_ref[...], w_ref[...],
                         preferred_element_type=jnp.float32,
                         precision=precision)


def _bf(x):
    return x.astype(jnp.bfloat16).astype(jnp.float32)


def _pick_block(n, cap):
    if n <= cap:
        return n
    best = None
    for b in range(128, cap + 1, 128):
        if n % b == 0:
            best = b
    if best is None:
        raise ValueError(f"no block for {n}")
    return best


def _mm(x, w, exact=False):
    M, K = x.shape
    _, N = w.shape
    bm = 544 if M % 544 == 0 else M
    bn = _pick_block(N, 1024) if N >= 128 else N
    grid = (M // bm, N // bn)
    prec = (jax.lax.Precision.HIGHEST if exact
            else jax.lax.Precision.DEFAULT)
    return pl.pallas_call(
        functools.partial(_mm_kernel, precision=prec),
        grid=grid,
        in_specs=[pl.BlockSpec((bm, K), lambda i, j: (i, 0)),
                  pl.BlockSpec((K, bn), lambda i, j: (0, j))],
        out_specs=pl.BlockSpec((bm, bn), lambda i, j: (i, j)),
        out_shape=jax.ShapeDtypeStruct((M, N), jnp.float32),
    )(x, w)


def _linear(p, x, exact=False):
    K, N = p["W"].shape
    return _mm(x, p["W"], exact=(exact or K <= 2)) + p["b"]


def _gat_conv(p, x, src, dst, num_nodes, hdn2=None):
    heads, out_ch = _HEADS, _FD
    if hdn2 is None:
        hdn2 = _mm(x, p["W"])
    hdn = hdn2.reshape(num_nodes, heads, out_ch)
    eyeh = jnp.eye(heads, dtype=jnp.float32)
    amat = jnp.zeros((heads, out_ch, 128), jnp.float32)
    amat = amat.at[:, :, :heads].set(p["a_src"][:, :, None] * eyeh[:, None, :])
    amat = amat.at[:, :, 8:8 + heads].set(
        p["a_dst"][:, :, None] * eyeh[:, None, :])
    ad_pack = _mm(hdn2, amat.reshape(heads * out_ch, 128), exact=True)
    a_s = ad_pack[:, :heads]
    a_d = ad_pack[:, 8:8 + heads]

    e = jax.nn.leaky_relu(a_s[src] + a_d[dst], 0.2)
    emax = jax.ops.segment_max(e, dst, num_segments=num_nodes)
    emax = jnp.where(jnp.isfinite(emax), emax, 0.0)
    w = jnp.exp(e - emax[dst])
    denom = jax.ops.segment_sum(w, dst, num_segments=num_nodes)
    msg = hdn[src] * w[:, :, None]
    out = jax.ops.segment_sum(msg, dst, num_segments=num_nodes)
    out = out / (denom[:, :, None] + 1e-16)
    return out.reshape(num_nodes, heads * out_ch) + p["b"]


def _conv2d(p, x):
    y = jax.lax.conv_general_dilated(
        x, p["W"], (1, 1), "SAME",
        dimension_numbers=("NCHW", "OIHW", "NCHW"))
    return y + p["b"][None, :, None, None]


def _maxpool2(x):
    return jax.lax.reduce_window(x, -jnp.inf, jax.lax.max,
                                 (1, 1, 2, 2), (1, 1, 2, 2), "VALID")


def _cnn_encode(params, x):
    for nm in ["c1", "c2", "c3", "c4"]:
        x = _maxpool2(jax.nn.relu(_conv2d(params[nm], x)))
    x = x.reshape(x.shape[0], -1)
    return _linear(params["linear_bottleneck"], x, exact=True)


def kernel(h, edge_index, node_pos, node_size, iou, batch, ptr,
           block_condition, params):
    relu = jax.nn.relu
    src = edge_index[0]
    dst = edge_index[1]

    h_iou = relu(_linear(params["enc_iou"], iou))
    h_exist = _linear(params["exist_emb"], h)
    h_pos = relu(_linear(params["pos_emb"], node_pos))
    h_size = relu(_linear(params["size_emb"], node_size))
    w_fe = params["feature_emb"]["W"]
    ft = relu(_mm(h_exist, w_fe[:h_exist.shape[1]], exact=True)
              + jnp.tile(_bf(w_fe[h_exist.shape[1]:]), (_B, 1))
              + params["feature_emb"]["b"])
    input_ft = jnp.concatenate([h_iou, h_size, h_pos, ft], 1)

    w_e1 = params["e_conv1"]["W"]
    hdn2_e1 = (_mm(h_iou, w_e1[:64], exact=True)
               + _mm(h_size, w_e1[64:192], exact=True)
               + _mm(h_pos, w_e1[192:320], exact=True)
               + _mm(ft, w_e1[320:]))
    n1 = relu(_gat_conv(params["e_conv1"], None, src, dst, _NT,
                        hdn2=hdn2_e1))
    n2 = relu(_gat_conv(params["e_conv2"], n1, src, dst, _NT))
    n3 = relu(_gat_conv(params["e_conv3"], n2, src, dst, _NT))

    g0 = jax.ops.segment_max(input_ft, batch, num_segments=_B)
    g1 = jax.ops.segment_max(n1, batch, num_segments=_B)
    g2 = jax.ops.segment_max(n2, batch, num_segments=_B)
    g3 = jax.ops.segment_max(n3, batch, num_segments=_B)
    latent = _linear(params["aggregate"],
                     jnp.concatenate([g0, g1, g2, g3], 1))
    mu = _linear(params["z_mu"], latent)
    log_var = _linear(params["z_var"], latent)
    eps = jax.random.normal(jax.random.key(1), mu.shape, jnp.float32)
    z = jnp.exp(0.5 * log_var) * eps + mu
    cond = _cnn_encode(params, block_condition)
    zc = jnp.concatenate([z, cond], 1)
    zf = _linear(params["dec_feature_init"], zc).reshape(_B * _N, -1)
    w_d1 = params["d_conv1"]["W"]
    hdn2_d1 = (_mm(relu(zf), w_d1[:_FD])
               + jnp.tile(_bf(w_d1[_FD:]), (_B, 1)))
    d1 = relu(_gat_conv(params["d_conv1"], None, src, dst, _NT,
                        hdn2=hdn2_d1))
    d2 = relu(_gat_conv(params["d_conv2"], d1, src, dst, _NT))
    d3 = relu(_gat_conv(params["d_conv3"], d2, src, dst, _NT))

    exist = _linear(params["dec_exist"], d3)
    pos_x = _linear(params["out_pos_x"], relu(_linear(params["dec_pos_x"], d3)))
    pos_y = _linear(params["out_pos_y"], relu(_linear(params["dec_pos_y"], d3)))
    height = _linear(params["out_height"],
                     relu(_linear(params["dec_height"], d3)))
    width = _linear(params["out_width"], relu(_linear(params["dec_width"], d3)))
    iou_out = _linear(params["out_iou"], relu(_linear(params["dec_iou"], d3)))
    pos = jnp.concatenate([pos_x, pos_y], 1)
    size = jnp.concatenate([height, width], 1)
    return exist, pos, size, mu, log_var, iou_out
